# Initial kernel scaffold; baseline (speedup 1.0000x reference)
import jax
import jax.numpy as jnp
from jax.experimental import pallas as pl


def kernel(up1_p2, up1_p3, up2_p2, up2_p3):
    raise NotImplementedError("write your pallas kernel here")


# speedup vs baseline: 1.6029x; 1.6029x over previous
import functools

import jax
import jax.numpy as jnp
from jax.experimental import pallas as pl
from jax.experimental.pallas import tpu as pltpu


def _ntxent_kernel(z1_ref, z2_ref, out_ref, *, B, C, temperature):
    f32 = jnp.float32
    inv_t = f32(1.0 / temperature)

    s1 = [z1_ref[b * C:(b + 1) * C, :] for b in range(B)]
    s2 = [z2_ref[b * C:(b + 1) * C, :] for b in range(B)]

    def rdot(a, b):
        return jnp.sum(a * b, axis=0, keepdims=True)

    invn = [jax.lax.rsqrt(jnp.maximum(rdot(s, s), f32(1e-24)))
            for s in (s1 + s2)]

    def sim(ra, rb, ia, ib):
        return (rdot(ra, rb) * (invn[ia] * invn[ib])) * inv_t

    t_ij = [[sim(s1[a], s2[b], a, B + b) for b in range(B)] for a in range(B)]
    e_ij = [[jnp.exp(t) for t in row] for row in t_ij]

    e_ii = {}
    e_jj = {}
    for a in range(B):
        for b in range(a + 1, B):
            e_ii[(a, b)] = jnp.exp(sim(s1[a], s1[b], a, b))
            e_jj[(a, b)] = jnp.exp(sim(s2[a], s2[b], B + a, B + b))

    def tsum(vals):
        acc = vals[0]
        for v in vals[1:]:
            acc = acc + v
        return acc

    loss = None
    pos = None
    for a in range(B):
        den_i = tsum([e_ii[(min(a, b), max(a, b))] for b in range(B) if b != a]
                     + e_ij[a])
        den_j = tsum([e_jj[(min(a, b), max(a, b))] for b in range(B) if b != a]
                     + [e_ij[r][a] for r in range(B)])
        term = jnp.log(den_i) + jnp.log(den_j)
        loss = term if loss is None else loss + term
        pos = t_ij[a][a] if pos is None else pos + t_ij[a][a]

    loss = (loss - 2.0 * pos) * f32(1.0 / (2 * B))
    out_ref[...] = loss.astype(out_ref.dtype)


def _round_up(x, m):
    return ((x + m - 1) // m) * m


def _per_position_losses(z1, z2, B, C, temperature):
    BC, P = z1.shape
    p_pad = _round_up(P, 128)
    tp = 512
    while tp > 128 and (p_pad % tp != 0 or p_pad // tp < 2):
        tp //= 2
    p_pad = _round_up(p_pad, tp)
    if p_pad != P:
        z1 = jnp.pad(z1, ((0, 0), (0, p_pad - P)))
        z2 = jnp.pad(z2, ((0, 0), (0, p_pad - P)))
    grid = (p_pad // tp,)
    out = pl.pallas_call(
        functools.partial(_ntxent_kernel, B=B, C=C,
                          temperature=float(temperature)),
        out_shape=jax.ShapeDtypeStruct((1, p_pad), jnp.float32),
        grid=grid,
        in_specs=[
            pl.BlockSpec((BC, tp), lambda p: (0, p)),
            pl.BlockSpec((BC, tp), lambda p: (0, p)),
        ],
        out_specs=pl.BlockSpec((1, tp), lambda p: (0, p)),
        compiler_params=pltpu.CompilerParams(
            dimension_semantics=("parallel",),
            vmem_limit_bytes=64 << 20,
        ),
    )(z1, z2)
    return out[:, :P]


def kernel(up1_p2, up1_p3, up2_p2, up2_p3):
    temperature = 0.5
    total_sum = jnp.float32(0.0)
    total_count = 0
    for x1, x2 in ((up1_p2, up2_p2), (up1_p3, up2_p3)):
        B, C = int(x1.shape[0]), int(x1.shape[1])
        z1 = x1.reshape(B * C, -1)
        z2 = x2.reshape(B * C, -1)
        P = z1.shape[1]
        per_pos = _per_position_losses(z1, z2, B, C, temperature)
        total_sum = total_sum + jnp.sum(per_pos[0, :])
        total_count += P
    return total_sum / jnp.float32(total_count)
